# baseline (speedup 1.0000x reference)
import jax
import jax.numpy as jnp
from jax import lax
from jax.experimental import pallas as pl
from jax.experimental.pallas import tpu as pltpu
from jax.experimental.pallas import tpu_sc as plsc

_N_NODES = 100000
_DEG = 32
_D = 128
_H = 32
_SEEDS = 1024
_K1 = 25
_K2 = 10
_B1 = _SEEDS * _K1
_B2 = _B1 * _K2

_NCORE = 2
_NSUB = 16
_NW = _NCORE * _NSUB

_SC_PARAMS = dict(
    compiler_params=pltpu.CompilerParams(
        needs_layout_passes=False, use_tc_tiling_on_sc=False),
)


def _sample_cols():
    key = jax.random.key(42)
    p0 = jax.random.permutation(jax.random.fold_in(key, 0), _DEG)[:_K1].astype(jnp.int32)
    p1 = jax.random.permutation(jax.random.fold_in(key, 1), _DEG)[:_K2].astype(jnp.int32)
    c1 = jnp.zeros((32,), jnp.int32).at[:_K1].set(p0)
    c2 = jnp.zeros((16,), jnp.int32).at[:_K2].set(p1)
    return c1, c2


def _sc_mesh():
    return plsc.VectorSubcoreMesh(core_axis_name="c", subcore_axis_name="s")


def _wid():
    return lax.axis_index("s") * _NCORE + lax.axis_index("c")


def _expand_body(ids_hbm, adjflat_hbm, c1_hbm, c2_hbm, out1_hbm, out2_hbm,
                 ids_v, cols_v, idx1_v, val1_v, idx2_v, val2_v, sem):
    n = _SEEDS // _NW
    base = _wid() * n
    pltpu.sync_copy(ids_hbm.at[pl.ds(base, n)], ids_v)
    pltpu.sync_copy(c1_hbm, cols_v.at[pl.ds(0, 32)])
    pltpu.sync_copy(c2_hbm, cols_v.at[pl.ds(32, 16)])
    cv1 = [cols_v[pl.ds(0, 16)], cols_v[pl.ds(16, 16)]]
    lanes = lax.iota(jnp.int32, 16)

    for k in range(_K1):
        c = cv1[k // 16][k % 16]
        for j in range(n // 16):
            idx1_v[pl.ds(k * n + j * 16, 16)] = ids_v[pl.ds(j * 16, 16)] * _DEG + c
    h1 = []
    for c in range(_K1 * n // 80):
        h1.append(pltpu.async_copy(
            adjflat_hbm.at[idx1_v.at[pl.ds(c * 80, 80)]], val1_v.at[pl.ds(c * 80, 80)], sem))
    for h in h1:
        h.wait()
    for k in range(_K1):
        pltpu.sync_copy(val1_v.at[pl.ds(k * n, n)],
                        out1_hbm.at[pl.ds(k * _SEEDS + base, n)])

    ck2 = [plsc.load_gather(cols_v, [(lanes + p * 16) % _K2 + 32]) for p in range(5)]
    for g in range(_K1 * n * _K2 // 16):
        mv = (lanes + g * 16) // _K2
        idv = plsc.load_gather(val1_v, [mv])
        idx2_v[pl.ds(g * 16, 16)] = idv * _DEG + ck2[g % 5]
    h2 = []
    for r in range(_K1 * n * _K2 // 80):
        h2.append(pltpu.async_copy(
            adjflat_hbm.at[idx2_v.at[pl.ds(r * 80, 80)]], val2_v.at[pl.ds(r * 80, 80)], sem))
    for h in h2:
        h.wait()
    for k in range(_K1):
        pltpu.sync_copy(val2_v.at[pl.ds(k * n * _K2, n * _K2)],
                        out2_hbm.at[pl.ds((k * _SEEDS + base) * _K2, n * _K2)])


def _expand(ids, adjflat, c1, c2):
    n = _SEEDS // _NW
    f = pl.kernel(
        _expand_body,
        out_type=(jax.ShapeDtypeStruct((_B1,), jnp.int32),
                  jax.ShapeDtypeStruct((_B2,), jnp.int32)),
        mesh=_sc_mesh(),
        scratch_types=[
            pltpu.VMEM((n,), jnp.int32),
            pltpu.VMEM((128,), jnp.int32),
            pltpu.VMEM((_K1 * n,), jnp.int32),
            pltpu.VMEM((_K1 * n,), jnp.int32),
            pltpu.VMEM((_K1 * n * _K2,), jnp.int32),
            pltpu.VMEM((_K1 * n * _K2,), jnp.int32),
            pltpu.SemaphoreType.DMA,
        ],
        **_SC_PARAMS,
    )
    return f(ids, adjflat, c1, c2)


def _gatherf_body(feats_hbm, ids_hbm, ids1_hbm, f0_out, f1_out,
                  idx_v, f0_v, f1_v, sem):
    n0 = _SEEDS // _NW
    n1 = _B1 // _NW
    b0 = _wid() * n0
    b1 = _wid() * n1
    pltpu.sync_copy(ids_hbm.at[pl.ds(b0, n0)], idx_v.at[pl.ds(0, n0)])
    pltpu.sync_copy(ids1_hbm.at[pl.ds(b1, n1)], idx_v.at[pl.ds(n0, n1)])
    handles = [pltpu.async_copy(feats_hbm.at[idx_v.at[pl.ds(0, n0)]], f0_v, sem)]
    for c in range(n1 // 80):
        handles.append(pltpu.async_copy(
            feats_hbm.at[idx_v.at[pl.ds(n0 + c * 80, 80)]],
            f1_v.at[pl.ds(c * 80, 80)], sem))
    for h in handles:
        h.wait()
    pltpu.sync_copy(f0_v, f0_out.at[pl.ds(b0, n0), :])
    pltpu.sync_copy(f1_v, f1_out.at[pl.ds(b1, n1), :])


def _gatherf(feats, ids, ids1):
    n0 = _SEEDS // _NW
    n1 = _B1 // _NW
    f = pl.kernel(
        _gatherf_body,
        out_type=(jax.ShapeDtypeStruct((_SEEDS, _D), jnp.float32),
                  jax.ShapeDtypeStruct((_B1, _D), jnp.float32)),
        mesh=_sc_mesh(),
        scratch_types=[
            pltpu.VMEM((n0 + n1,), jnp.int32),
            pltpu.VMEM((n0, _D), jnp.float32),
            pltpu.VMEM((n1, _D), jnp.float32),
            pltpu.SemaphoreType.DMA,
        ],
        **_SC_PARAMS,
    )
    return f(feats, ids, ids1)


_P = 16
_E = _P * _K2
_NCH = (_B1 // _NW) // _P


def _mega_body(feats_hbm, a_hbm, ids_hbm, ids1_hbm, ids2_hbm,
               ag1_out, g0_out, g1_out,
               ids1_v, idx2_v, a1_v, g0_v, a2_v, f2_v, s_v, ws_v, acc_v,
               sem, osem0, osem1):
    n0 = _SEEDS // _NW
    n1 = _B1 // _NW
    b0 = _wid() * n0
    b1 = _wid() * n1
    lanes = lax.iota(jnp.int32, 16)
    osems = [osem0, osem1]

    pltpu.sync_copy(ids1_hbm.at[pl.ds(b1, n1)], ids1_v)
    pltpu.sync_copy(ids2_hbm.at[pl.ds(b1 * _K2, n1 * _K2)], idx2_v)
    hg = []
    for c in range(n1 // 80):
        hg.append(pltpu.async_copy(
            a_hbm.at[ids1_v.at[pl.ds(c * 80, 80)]], a1_v.at[pl.ds(c * 80, 80)], sem))
    for h in hg:
        h.wait()
    pltpu.sync_copy(a1_v, g1_out.at[pl.ds(b1, n1), :])
    pltpu.sync_copy(ids_hbm.at[pl.ds(b0, n0)], ids1_v.at[pl.ds(0, n0)])
    pltpu.async_copy(a_hbm.at[ids1_v.at[pl.ds(0, n0)]], g0_v, sem).wait()
    pltpu.sync_copy(g0_v, g0_out.at[pl.ds(b0, n0), :])

    ploc = [(lanes + g * 16) // _K2 for g in range(_E // 16)]

    def fire(c, buf):
        for sub in range(_E // 80):
            pltpu.async_copy(
                a_hbm.at[idx2_v.at[pl.ds(c * _E + sub * 80, 80)]],
                a2_v.at[pl.ds(buf * _E + sub * 80, 80)], sem)
            pltpu.async_copy(
                feats_hbm.at[idx2_v.at[pl.ds(c * _E + sub * 80, 80)]],
                f2_v.at[pl.ds(buf * _E + sub * 80, 80)], sem)

    def drain_in(buf):
        pltpu.make_async_copy(
            a_hbm.at[pl.ds(0, _E)], a2_v.at[pl.ds(buf * _E, _E)], sem).wait()
        pltpu.make_async_copy(
            feats_hbm.at[pl.ds(0, _E)], f2_v.at[pl.ds(buf * _E, _E)], sem).wait()

    def drain_out(buf):
        pltpu.make_async_copy(
            feats_hbm.at[pl.ds(0, _P)], acc_v.at[pl.ds(buf * _P, _P)],
            osems[buf]).wait()

    def compute(c, buf):
        eb = buf * _E
        for g in range(_E // 16):
            acc = jnp.zeros((16,), jnp.float32)
            p_glob = ploc[g] + c * _P
            e_loc = lanes + eb + g * 16
            for h in range(_H):
                hv = jnp.full((16,), h, jnp.int32)
                a2h = plsc.load_gather(a2_v, [e_loc, hv])
                a1h = plsc.load_gather(a1_v, [p_glob, hv])
                acc = acc + a2h * a1h
            s_v[pl.ds(g * 16, 16)] = acc
        sk = [plsc.load_gather(s_v, [lanes * _K2 + k]) for k in range(_K2)]
        m = sk[0]
        for k in range(1, _K2):
            m = jnp.maximum(m, sk[k])
        ek = [jnp.exp(x - m) for x in sk]
        den = ek[0]
        for k in range(1, _K2):
            den = den + ek[k]
        inv = 1.0 / den
        for k in range(_K2):
            plsc.store_scatter(ws_v, [lanes, jnp.full((16,), k, jnp.int32)], ek[k] * inv)

        @pl.when(c >= 2)
        def _():
            drain_out(buf)

        def body(i, _):
            wsrow = ws_v[i, pl.ds(0, 16)]
            acc = [jnp.zeros((16,), jnp.float32) for _ in range(_D // 16)]
            for k in range(_K2):
                w = wsrow[k]
                for v in range(_D // 16):
                    acc[v] = acc[v] + w * f2_v[eb + i * _K2 + k, pl.ds(v * 16, 16)]
            for v in range(_D // 16):
                acc_v[buf * _P + i, pl.ds(v * 16, 16)] = acc[v]
            return 0

        lax.fori_loop(0, _P, body, 0)
        pltpu.async_copy(acc_v.at[pl.ds(buf * _P, _P)],
                         ag1_out.at[pl.ds(b1 + c * _P, _P), :], osems[buf])

    fire(0, 0)

    def pair(cc, _):
        c0 = cc * 2
        fire(c0 + 1, 1)
        drain_in(0)
        compute(c0, 0)

        @pl.when(c0 + 2 < _NCH)
        def _():
            fire(c0 + 2, 0)
        drain_in(1)
        compute(c0 + 1, 1)
        return 0

    lax.fori_loop(0, _NCH // 2, pair, 0)
    drain_out(0)
    drain_out(1)


def _mega(feats, a, ids, ids1, ids2):
    n1 = _B1 // _NW
    f = pl.kernel(
        _mega_body,
        out_type=(jax.ShapeDtypeStruct((_B1, _D), jnp.float32),
                  jax.ShapeDtypeStruct((_SEEDS, _H), jnp.float32),
                  jax.ShapeDtypeStruct((_B1, _H), jnp.float32)),
        mesh=_sc_mesh(),
        scratch_types=[
            pltpu.VMEM((n1,), jnp.int32),
            pltpu.VMEM((n1 * _K2,), jnp.int32),
            pltpu.VMEM((n1, _H), jnp.float32),
            pltpu.VMEM((_SEEDS // _NW, _H), jnp.float32),
            pltpu.VMEM((2 * _E, _H), jnp.float32),
            pltpu.VMEM((2 * _E, _D), jnp.float32),
            pltpu.VMEM((_E,), jnp.float32),
            pltpu.VMEM((_P, 16), jnp.float32),
            pltpu.VMEM((2 * _P, _D), jnp.float32),
            pltpu.SemaphoreType.DMA,
            pltpu.SemaphoreType.DMA,
            pltpu.SemaphoreType.DMA,
        ],
        **_SC_PARAMS,
    )
    return f(feats, a, ids, ids1, ids2)


def _amm_body(f_ref, w_ref, o_ref):
    o_ref[...] = jnp.dot(f_ref[...], w_ref[...], preferred_element_type=jnp.float32)


def _amm(feats, att_w1):
    blk = 2000
    return pl.pallas_call(
        _amm_body,
        grid=(_N_NODES // blk,),
        in_specs=[pl.BlockSpec((blk, _D), lambda i: (i, 0)),
                  pl.BlockSpec((_D, _H), lambda i: (0, 0))],
        out_specs=pl.BlockSpec((blk, _H), lambda i: (i, 0)),
        out_shape=jax.ShapeDtypeStruct((_N_NODES, _H), jnp.float32),
    )(feats, att_w1)


def _tail_body(f0_ref, g0_ref, f1_ref, g1r_ref, ag1_ref,
               axw1_ref, anw1_ref, aw2_ref, axw2_ref, anw2_ref,
               fcw_ref, fcb_ref, o_ref):
    F0 = f0_ref[...]
    G0 = g0_ref[...]
    axw1 = axw1_ref[...]
    anw1 = anw1_ref[...]

    s = [jnp.sum(g1r_ref[k] * G0, axis=1, keepdims=True) for k in range(_K1)]
    m = s[0]
    for k in range(1, _K1):
        m = jnp.maximum(m, s[k])
    e = [jnp.exp(sk - m) for sk in s]
    den = e[0]
    for k in range(1, _K1):
        den = den + e[k]
    inv = 1.0 / den
    agg0 = jnp.zeros(F0.shape, jnp.float32)
    for k in range(_K1):
        agg0 = agg0 + (e[k] * inv) * f1_ref[k]

    g0a = jnp.maximum(F0 @ axw1, 0.0)
    g0b = jnp.maximum(agg0 @ anw1, 0.0)

    aw2a = aw2_ref[0:_D, :]
    aw2b = aw2_ref[_D:2 * _D, :]
    x2 = g0a @ aw2a + g0b @ aw2b

    g1a, g1b, s2 = [], [], []
    for k in range(_K1):
        a = jnp.maximum(f1_ref[k] @ axw1, 0.0)
        b = jnp.maximum(ag1_ref[k] @ anw1, 0.0)
        n2 = a @ aw2a + b @ aw2b
        g1a.append(a)
        g1b.append(b)
        s2.append(jnp.sum(n2 * x2, axis=1, keepdims=True))
    m2 = s2[0]
    for k in range(1, _K1):
        m2 = jnp.maximum(m2, s2[k])
    e2 = [jnp.exp(sk - m2) for sk in s2]
    den2 = e2[0]
    for k in range(1, _K1):
        den2 = den2 + e2[k]
    inv2 = 1.0 / den2
    agg2a = jnp.zeros(F0.shape, jnp.float32)
    agg2b = jnp.zeros(F0.shape, jnp.float32)
    for k in range(_K1):
        w = e2[k] * inv2
        agg2a = agg2a + w * g1a[k]
        agg2b = agg2b + w * g1b[k]

    h0a = jnp.maximum(g0a @ axw2_ref[0:_D, :] + g0b @ axw2_ref[_D:2 * _D, :], 0.0)
    h0b = jnp.maximum(agg2a @ anw2_ref[0:_D, :] + agg2b @ anw2_ref[_D:2 * _D, :], 0.0)
    nrm = jnp.sqrt(jnp.sum(h0a * h0a, axis=1, keepdims=True)
                   + jnp.sum(h0b * h0b, axis=1, keepdims=True))
    sc = 1.0 / jnp.maximum(nrm, 1e-12)
    o_ref[...] = (h0a * sc) @ fcw_ref[0:_D, :] + (h0b * sc) @ fcw_ref[_D:2 * _D, :] + fcb_ref[...]


def _tail(F0, G0, F1t, G1t, AG1t, fcx_w1, fcn_w1, att_w2, fcx_w2, fcn_w2, fc_w, fc_b):
    S = 256
    nc = fc_w.shape[1]
    return pl.pallas_call(
        _tail_body,
        grid=(_SEEDS // S,),
        in_specs=[
            pl.BlockSpec((S, _D), lambda i: (i, 0)),
            pl.BlockSpec((S, _H), lambda i: (i, 0)),
            pl.BlockSpec((_K1, S, _D), lambda i: (0, i, 0)),
            pl.BlockSpec((_K1, S, _H), lambda i: (0, i, 0)),
            pl.BlockSpec((_K1, S, _D), lambda i: (0, i, 0)),
            pl.BlockSpec((_D, _D), lambda i: (0, 0)),
            pl.BlockSpec((_D, _D), lambda i: (0, 0)),
            pl.BlockSpec((2 * _D, _H), lambda i: (0, 0)),
            pl.BlockSpec((2 * _D, _D), lambda i: (0, 0)),
            pl.BlockSpec((2 * _D, _D), lambda i: (0, 0)),
            pl.BlockSpec((2 * _D, nc), lambda i: (0, 0)),
            pl.BlockSpec((1, nc), lambda i: (0, 0)),
        ],
        out_specs=pl.BlockSpec((S, nc), lambda i: (i, 0)),
        out_shape=jax.ShapeDtypeStruct((_SEEDS, nc), jnp.float32),
    )(F0, G0, F1t, G1t, AG1t, fcx_w1, fcn_w1, att_w2, fcx_w2, fcn_w2, fc_w, fc_b)


def kernel(ids, feats, adj, att_w1, fcx_w1, fcn_w1, att_w2, fcx_w2, fcn_w2, fc_w, fc_b):
    ids = ids.astype(jnp.int32)
    adjflat = adj.astype(jnp.int32).reshape(-1)
    cols1, cols2 = _sample_cols()

    ids1, ids2 = _expand(ids, adjflat, cols1, cols2)
    A = _amm(feats, att_w1)
    F0, F1 = _gatherf(feats, ids, ids1)
    AG1, G0, G1 = _mega(feats, A, ids, ids1, ids2)

    return _tail(F0, G0,
                 F1.reshape(_K1, _SEEDS, _D),
                 G1.reshape(_K1, _SEEDS, _H),
                 AG1.reshape(_K1, _SEEDS, _D),
                 fcx_w1, fcn_w1, att_w2, fcx_w2, fcn_w2, fc_w,
                 fc_b.reshape(1, -1))


# speedup vs baseline: 1.4829x; 1.4829x over previous
import jax
import jax.numpy as jnp
from jax import lax
from jax.experimental import pallas as pl
from jax.experimental.pallas import tpu as pltpu
from jax.experimental.pallas import tpu_sc as plsc

_N_NODES = 100000
_DEG = 32
_D = 128
_H = 32
_SEEDS = 1024
_K1 = 25
_K2 = 10
_B1 = _SEEDS * _K1
_B2 = _B1 * _K2

_NCORE = 2
_NSUB = 16
_NW = _NCORE * _NSUB

_SC_LINEAR = dict(
    compiler_params=pltpu.CompilerParams(
        needs_layout_passes=False, use_tc_tiling_on_sc=False),
)
_SC_TILED = dict(
    compiler_params=pltpu.CompilerParams(
        needs_layout_passes=False, use_tc_tiling_on_sc=True),
)


def _sample_cols():
    key = jax.random.key(42)
    p0 = jax.random.permutation(jax.random.fold_in(key, 0), _DEG)[:_K1].astype(jnp.int32)
    p1 = jax.random.permutation(jax.random.fold_in(key, 1), _DEG)[:_K2].astype(jnp.int32)
    c1 = jnp.zeros((32,), jnp.int32).at[:_K1].set(p0)
    c2 = jnp.zeros((16,), jnp.int32).at[:_K2].set(p1)
    return c1, c2


def _sc_mesh():
    return plsc.VectorSubcoreMesh(core_axis_name="c", subcore_axis_name="s")


def _wid():
    return lax.axis_index("s") * _NCORE + lax.axis_index("c")


def _expand_body(ids_hbm, adjflat_hbm, c1_hbm, c2_hbm, out1_hbm, out2_hbm,
                 ids_v, cols_v, idx1_v, val1_v, idx2_v, val2_v, sem):
    n = _SEEDS // _NW
    base = _wid() * n
    pltpu.sync_copy(ids_hbm.at[pl.ds(base, n)], ids_v)
    pltpu.sync_copy(c1_hbm, cols_v.at[pl.ds(0, 32)])
    pltpu.sync_copy(c2_hbm, cols_v.at[pl.ds(32, 16)])
    cv1 = [cols_v[pl.ds(0, 16)], cols_v[pl.ds(16, 16)]]
    lanes = lax.iota(jnp.int32, 16)

    for k in range(_K1):
        c = cv1[k // 16][k % 16]
        for j in range(n // 16):
            idx1_v[pl.ds(k * n + j * 16, 16)] = ids_v[pl.ds(j * 16, 16)] * _DEG + c
    h1 = []
    for c in range(_K1 * n // 80):
        h1.append(pltpu.async_copy(
            adjflat_hbm.at[idx1_v.at[pl.ds(c * 80, 80)]], val1_v.at[pl.ds(c * 80, 80)], sem))
    for h in h1:
        h.wait()
    for k in range(_K1):
        pltpu.sync_copy(val1_v.at[pl.ds(k * n, n)],
                        out1_hbm.at[pl.ds(k * _SEEDS + base, n)])

    ck2 = [plsc.load_gather(cols_v, [(lanes + p * 16) % _K2 + 32]) for p in range(5)]
    for g in range(_K1 * n * _K2 // 16):
        mv = (lanes + g * 16) // _K2
        idv = plsc.load_gather(val1_v, [mv])
        idx2_v[pl.ds(g * 16, 16)] = idv * _DEG + ck2[g % 5]
    h2 = []
    for r in range(_K1 * n * _K2 // 80):
        h2.append(pltpu.async_copy(
            adjflat_hbm.at[idx2_v.at[pl.ds(r * 80, 80)]], val2_v.at[pl.ds(r * 80, 80)], sem))
    for h in h2:
        h.wait()
    for k in range(_K1):
        pltpu.sync_copy(val2_v.at[pl.ds(k * n * _K2, n * _K2)],
                        out2_hbm.at[pl.ds((k * _SEEDS + base) * _K2, n * _K2)])


def _expand(ids, adjflat, c1, c2):
    n = _SEEDS // _NW
    f = pl.kernel(
        _expand_body,
        out_type=(jax.ShapeDtypeStruct((_B1,), jnp.int32),
                  jax.ShapeDtypeStruct((_B2,), jnp.int32)),
        mesh=_sc_mesh(),
        scratch_types=[
            pltpu.VMEM((n,), jnp.int32),
            pltpu.VMEM((128,), jnp.int32),
            pltpu.VMEM((_K1 * n,), jnp.int32),
            pltpu.VMEM((_K1 * n,), jnp.int32),
            pltpu.VMEM((_K1 * n * _K2,), jnp.int32),
            pltpu.VMEM((_K1 * n * _K2,), jnp.int32),
            pltpu.SemaphoreType.DMA,
        ],
        **_SC_LINEAR,
    )
    return f(ids, adjflat, c1, c2)


def _gatherf_body(feats_hbm, ids_hbm, ids1_hbm, f0_out, f1_out,
                  idx_v, f0_v, f1_v, sem):
    n0 = _SEEDS // _NW
    n1 = _B1 // _NW
    b0 = _wid() * n0
    b1 = _wid() * n1
    pltpu.sync_copy(ids_hbm.at[pl.ds(b0, n0)], idx_v.at[pl.ds(0, n0)])
    pltpu.sync_copy(ids1_hbm.at[pl.ds(b1, n1)], idx_v.at[pl.ds(n0, n1)])
    handles = [pltpu.async_copy(feats_hbm.at[idx_v.at[pl.ds(0, n0)]], f0_v, sem)]
    for c in range(n1 // 80):
        handles.append(pltpu.async_copy(
            feats_hbm.at[idx_v.at[pl.ds(n0 + c * 80, 80)]],
            f1_v.at[pl.ds(c * 80, 80)], sem))
    for h in handles:
        h.wait()
    pltpu.sync_copy(f0_v, f0_out.at[pl.ds(b0, n0), :])
    pltpu.sync_copy(f1_v, f1_out.at[pl.ds(b1, n1), :])


def _gatherf(feats, ids, ids1):
    n0 = _SEEDS // _NW
    n1 = _B1 // _NW
    f = pl.kernel(
        _gatherf_body,
        out_type=(jax.ShapeDtypeStruct((_SEEDS, _D), jnp.float32),
                  jax.ShapeDtypeStruct((_B1, _D), jnp.float32)),
        mesh=_sc_mesh(),
        scratch_types=[
            pltpu.VMEM((n0 + n1,), jnp.int32),
            pltpu.VMEM((n0, _D), jnp.float32),
            pltpu.VMEM((n1, _D), jnp.float32),
            pltpu.SemaphoreType.DMA,
        ],
        **_SC_TILED,
    )
    return f(feats, ids, ids1)


_P = 16
_E = _P * _K2
_NCH = (_B1 // _NW) // _P


def _mega_body(feats_hbm, a_hbm, ids_hbm, ids1_hbm, ids2_hbm,
               ag1_out, g0_out, g1_out,
               ids1_v, idx2_v, a1c_v, g0_v, a2_v, f2_v, s_v, ws_v, acc_v,
               sem, gsem0, gsem1, osem0, osem1):
    n0 = _SEEDS // _NW
    n1 = _B1 // _NW
    b0 = _wid() * n0
    b1 = _wid() * n1
    lanes = lax.iota(jnp.int32, 16)
    gsems = [gsem0, gsem1]
    osems = [osem0, osem1]

    pltpu.sync_copy(ids1_hbm.at[pl.ds(b1, n1)], ids1_v)
    pltpu.sync_copy(ids2_hbm.at[pl.ds(b1 * _K2, n1 * _K2)], idx2_v)
    pltpu.sync_copy(ids_hbm.at[pl.ds(b0, n0)], ids1_v.at[pl.ds(n1 - n0, n0)])
    pltpu.async_copy(a_hbm.at[ids1_v.at[pl.ds(n1 - n0, n0)]], g0_v, sem).wait()
    pltpu.sync_copy(ids1_hbm.at[pl.ds(b1 + n1 - n0, n0)], ids1_v.at[pl.ds(n1 - n0, n0)])
    pltpu.sync_copy(g0_v, g0_out.at[pl.ds(b0, n0), :])

    ploc = [(lanes + g * 16) // _K2 for g in range(_E // 16)]
    hrot = [(lanes + h0) % _H for h0 in range(_H)]

    def fire(c, buf):
        pltpu.async_copy(a_hbm.at[ids1_v.at[pl.ds(c * _P, _P)]],
                         a1c_v.at[pl.ds(buf * _P, _P)], sem)
        for sub in range(_E // 80):
            pltpu.async_copy(
                a_hbm.at[idx2_v.at[pl.ds(c * _E + sub * 80, 80)]],
                a2_v.at[pl.ds(buf * _E + sub * 80, 80)], sem)
            pltpu.async_copy(
                feats_hbm.at[idx2_v.at[pl.ds(c * _E + sub * 80, 80)]],
                f2_v.at[pl.ds(buf * _E + sub * 80, 80)], sem)

    def drain_in(buf):
        pltpu.make_async_copy(
            a_hbm.at[pl.ds(0, _P)], a1c_v.at[pl.ds(buf * _P, _P)], sem).wait()
        pltpu.make_async_copy(
            a_hbm.at[pl.ds(0, _E)], a2_v.at[pl.ds(buf * _E, _E)], sem).wait()
        pltpu.make_async_copy(
            feats_hbm.at[pl.ds(0, _E)], f2_v.at[pl.ds(buf * _E, _E)], sem).wait()

    def compute(c, buf):
        eb = buf * _E
        pb = buf * _P
        pltpu.async_copy(a1c_v.at[pl.ds(pb, _P)],
                         g1_out.at[pl.ds(b1 + c * _P, _P), :], gsems[buf])
        for g in range(_E // 16):
            acc = jnp.zeros((16,), jnp.float32)
            e_l = lanes + eb + g * 16
            p_l = ploc[g] + pb
            for h0 in range(_H):
                a2h = plsc.load_gather(a2_v, [e_l, hrot[h0]])
                a1h = plsc.load_gather(a1c_v, [p_l, hrot[h0]])
                acc = acc + a2h * a1h
            s_v[pl.ds(g * 16, 16)] = acc
        sk = [plsc.load_gather(s_v, [lanes * _K2 + k]) for k in range(_K2)]
        m = sk[0]
        for k in range(1, _K2):
            m = jnp.maximum(m, sk[k])
        ek = [jnp.exp(x - m) for x in sk]
        den = ek[0]
        for k in range(1, _K2):
            den = den + ek[k]
        inv = 1.0 / den
        for k in range(_K2):
            plsc.store_scatter(ws_v, [lanes, jnp.full((16,), k, jnp.int32)], ek[k] * inv)

        @pl.when(c >= 2)
        def _():
            pltpu.make_async_copy(
                feats_hbm.at[pl.ds(0, _P)], acc_v.at[pl.ds(pb, _P)],
                osems[buf]).wait()

        def body(i, _):
            wsrow = ws_v[i, pl.ds(0, 16)]
            acc = [jnp.zeros((16,), jnp.float32) for _ in range(_D // 16)]
            for k in range(_K2):
                w = wsrow[k]
                for v in range(_D // 16):
                    acc[v] = acc[v] + w * f2_v[eb + i * _K2 + k, pl.ds(v * 16, 16)]
            for v in range(_D // 16):
                acc_v[pb + i, pl.ds(v * 16, 16)] = acc[v]
            return 0

        lax.fori_loop(0, _P, body, 0)
        pltpu.async_copy(acc_v.at[pl.ds(pb, _P)],
                         ag1_out.at[pl.ds(b1 + c * _P, _P), :], osems[buf])

    def drain_g1(buf):
        pltpu.make_async_copy(
            feats_hbm.at[pl.ds(0, _P)], a1c_v.at[pl.ds(buf * _P, _P)],
            gsems[buf]).wait()

    fire(0, 0)

    def pair(cc, _):
        c0 = cc * 2

        @pl.when(c0 >= 1)
        def _():
            drain_g1(1)
        fire(c0 + 1, 1)
        drain_in(0)
        compute(c0, 0)

        @pl.when(c0 + 2 < _NCH)
        def _():
            drain_g1(0)
            fire(c0 + 2, 0)
        drain_in(1)
        compute(c0 + 1, 1)
        return 0

    lax.fori_loop(0, _NCH // 2, pair, 0)
    drain_g1(0)
    drain_g1(1)
    for b in range(2):
        pltpu.make_async_copy(
            feats_hbm.at[pl.ds(0, _P)], acc_v.at[pl.ds(b * _P, _P)],
            osems[b]).wait()


def _mega(feats, a, ids, ids1, ids2):
    n1 = _B1 // _NW
    f = pl.kernel(
        _mega_body,
        out_type=(jax.ShapeDtypeStruct((_B1, _D), jnp.float32),
                  jax.ShapeDtypeStruct((_SEEDS, _D), jnp.float32),
                  jax.ShapeDtypeStruct((_B1, _D), jnp.float32)),
        mesh=_sc_mesh(),
        scratch_types=[
            pltpu.VMEM((n1,), jnp.int32),
            pltpu.VMEM((n1 * _K2,), jnp.int32),
            pltpu.VMEM((2 * _P, _D), jnp.float32),
            pltpu.VMEM((_SEEDS // _NW, _D), jnp.float32),
            pltpu.VMEM((2 * _E, _D), jnp.float32),
            pltpu.VMEM((2 * _E, _D), jnp.float32),
            pltpu.VMEM((_E,), jnp.float32),
            pltpu.VMEM((_P, 16), jnp.float32),
            pltpu.VMEM((2 * _P, _D), jnp.float32),
            pltpu.SemaphoreType.DMA,
            pltpu.SemaphoreType.DMA,
            pltpu.SemaphoreType.DMA,
            pltpu.SemaphoreType.DMA,
            pltpu.SemaphoreType.DMA,
        ],
        **_SC_TILED,
    )
    return f(feats, a, ids, ids1, ids2)


def _amm_body(f_ref, w_ref, o_ref):
    o_ref[...] = jnp.dot(f_ref[...], w_ref[...], preferred_element_type=jnp.float32)


def _amm(feats, att_w1):
    blk = 2000
    w128 = jnp.pad(att_w1, ((0, 0), (0, _D - _H)))
    return pl.pallas_call(
        _amm_body,
        grid=(_N_NODES // blk,),
        in_specs=[pl.BlockSpec((blk, _D), lambda i: (i, 0)),
                  pl.BlockSpec((_D, _D), lambda i: (0, 0))],
        out_specs=pl.BlockSpec((blk, _D), lambda i: (i, 0)),
        out_shape=jax.ShapeDtypeStruct((_N_NODES, _D), jnp.float32),
    )(feats, w128)


def _tail_body(f0_ref, g0_ref, f1_ref, g1r_ref, ag1_ref,
               axw1_ref, anw1_ref, aw2_ref, axw2_ref, anw2_ref,
               fcw_ref, fcb_ref, o_ref):
    F0 = f0_ref[...]
    G0 = g0_ref[...]
    axw1 = axw1_ref[...]
    anw1 = anw1_ref[...]

    s = [jnp.sum(g1r_ref[k] * G0, axis=1, keepdims=True) for k in range(_K1)]
    m = s[0]
    for k in range(1, _K1):
        m = jnp.maximum(m, s[k])
    e = [jnp.exp(sk - m) for sk in s]
    den = e[0]
    for k in range(1, _K1):
        den = den + e[k]
    inv = 1.0 / den
    agg0 = jnp.zeros(F0.shape, jnp.float32)
    for k in range(_K1):
        agg0 = agg0 + (e[k] * inv) * f1_ref[k]

    g0a = jnp.maximum(F0 @ axw1, 0.0)
    g0b = jnp.maximum(agg0 @ anw1, 0.0)

    aw2a = aw2_ref[0:_D, :]
    aw2b = aw2_ref[_D:2 * _D, :]
    x2 = g0a @ aw2a + g0b @ aw2b

    g1a, g1b, s2 = [], [], []
    for k in range(_K1):
        a = jnp.maximum(f1_ref[k] @ axw1, 0.0)
        b = jnp.maximum(ag1_ref[k] @ anw1, 0.0)
        n2 = a @ aw2a + b @ aw2b
        g1a.append(a)
        g1b.append(b)
        s2.append(jnp.sum(n2 * x2, axis=1, keepdims=True))
    m2 = s2[0]
    for k in range(1, _K1):
        m2 = jnp.maximum(m2, s2[k])
    e2 = [jnp.exp(sk - m2) for sk in s2]
    den2 = e2[0]
    for k in range(1, _K1):
        den2 = den2 + e2[k]
    inv2 = 1.0 / den2
    agg2a = jnp.zeros(F0.shape, jnp.float32)
    agg2b = jnp.zeros(F0.shape, jnp.float32)
    for k in range(_K1):
        w = e2[k] * inv2
        agg2a = agg2a + w * g1a[k]
        agg2b = agg2b + w * g1b[k]

    h0a = jnp.maximum(g0a @ axw2_ref[0:_D, :] + g0b @ axw2_ref[_D:2 * _D, :], 0.0)
    h0b = jnp.maximum(agg2a @ anw2_ref[0:_D, :] + agg2b @ anw2_ref[_D:2 * _D, :], 0.0)
    nrm = jnp.sqrt(jnp.sum(h0a * h0a, axis=1, keepdims=True)
                   + jnp.sum(h0b * h0b, axis=1, keepdims=True))
    sc = 1.0 / jnp.maximum(nrm, 1e-12)
    o_ref[...] = (h0a * sc) @ fcw_ref[0:_D, :] + (h0b * sc) @ fcw_ref[_D:2 * _D, :] + fcb_ref[...]


def _tail(F0, G0, F1t, G1t, AG1t, fcx_w1, fcn_w1, att_w2, fcx_w2, fcn_w2, fc_w, fc_b):
    S = 256
    nc = fc_w.shape[1]
    return pl.pallas_call(
        _tail_body,
        grid=(_SEEDS // S,),
        in_specs=[
            pl.BlockSpec((S, _D), lambda i: (i, 0)),
            pl.BlockSpec((S, _D), lambda i: (i, 0)),
            pl.BlockSpec((_K1, S, _D), lambda i: (0, i, 0)),
            pl.BlockSpec((_K1, S, _D), lambda i: (0, i, 0)),
            pl.BlockSpec((_K1, S, _D), lambda i: (0, i, 0)),
            pl.BlockSpec((_D, _D), lambda i: (0, 0)),
            pl.BlockSpec((_D, _D), lambda i: (0, 0)),
            pl.BlockSpec((2 * _D, _H), lambda i: (0, 0)),
            pl.BlockSpec((2 * _D, _D), lambda i: (0, 0)),
            pl.BlockSpec((2 * _D, _D), lambda i: (0, 0)),
            pl.BlockSpec((2 * _D, nc), lambda i: (0, 0)),
            pl.BlockSpec((1, nc), lambda i: (0, 0)),
        ],
        out_specs=pl.BlockSpec((S, nc), lambda i: (i, 0)),
        out_shape=jax.ShapeDtypeStruct((_SEEDS, nc), jnp.float32),
    )(F0, G0, F1t, G1t, AG1t, fcx_w1, fcn_w1, att_w2, fcx_w2, fcn_w2, fc_w, fc_b)


def kernel(ids, feats, adj, att_w1, fcx_w1, fcn_w1, att_w2, fcx_w2, fcn_w2, fc_w, fc_b):
    ids = ids.astype(jnp.int32)
    adjflat = adj.astype(jnp.int32).reshape(-1)
    cols1, cols2 = _sample_cols()

    ids1, ids2 = _expand(ids, adjflat, cols1, cols2)
    A = _amm(feats, att_w1)
    F0, F1 = _gatherf(feats, ids, ids1)
    AG1, G0, G1 = _mega(feats, A, ids, ids1, ids2)

    return _tail(F0, G0,
                 F1.reshape(_K1, _SEEDS, _D),
                 G1.reshape(_K1, _SEEDS, _D),
                 AG1.reshape(_K1, _SEEDS, _D),
                 fcx_w1, fcn_w1, att_w2, fcx_w2, fcn_w2, fc_w,
                 fc_b.reshape(1, -1))
